# Initial kernel scaffold; baseline (speedup 1.0000x reference)
import jax
import jax.numpy as jnp
from jax.experimental import pallas as pl


def kernel(x, W1, b1, W2, b2, Wn, bn, W3, b3, W4, b4, Wnl, bnl):
    raise NotImplementedError("write your pallas kernel here")


# speedup vs baseline: 1.0958x; 1.0958x over previous
import functools

import jax
import jax.numpy as jnp
from jax.experimental import pallas as pl
from jax.experimental.pallas import tpu as pltpu

N_TOK = 8192
TOPK = 8
NEG_INF = float("-inf")
SPLIT = 592


def _chunk_dot(a_ref, w_ref, rows, kc):
    k = a_ref.shape[1]
    acc = None
    for c in range(k // kc):
        p = jax.lax.dot_general(
            a_ref[rows, c * kc:(c + 1) * kc], w_ref[:, c * kc:(c + 1) * kc],
            (((1,), (1,)), ((), ())), preferred_element_type=jnp.float32)
        acc = p if acc is None else acc + p
    return acc


def _mm_kernel(a_ref, w_ref, b_ref, o_ref, *, relu):
    i = pl.program_id(0)

    def compute(kc_top, kc_bot):
        top = _chunk_dot(a_ref, w_ref, slice(0, SPLIT), kc_top)
        bot = _chunk_dot(a_ref, w_ref, slice(SPLIT, a_ref.shape[0]), kc_bot)
        return jnp.concatenate([top, bot], axis=0)

    acc = jax.lax.cond(i % 2 == 0,
                       lambda: compute(512, 256),
                       lambda: compute(256, 512))
    acc = acc + b_ref[...]
    if relu:
        acc = jnp.maximum(acc, 0.0)
    o_ref[...] = acc


def _mm_kernel_narrow(a_ref, w_ref, b_ref, o_ref, *, relu):
    acc = jax.lax.dot_general(
        a_ref[...], w_ref[...], (((1,), (1,)), ((), ())),
        preferred_element_type=jnp.float32)
    acc = acc + b_ref[...]
    if relu:
        acc = jnp.maximum(acc, 0.0)
    o_ref[...] = acc


@functools.partial(jax.jit, static_argnames=("relu", "bm", "bn"))
def _mm(a, w, b, relu, bm=1024, bn=512):
    m, k = a.shape
    n = w.shape[0]
    if n < bn:
        bn = n
        body = functools.partial(_mm_kernel_narrow, relu=relu)
    else:
        body = functools.partial(_mm_kernel, relu=relu)
    grid = (m // bm, n // bn)
    return pl.pallas_call(
        body,
        grid=grid,
        in_specs=[
            pl.BlockSpec((bm, k), lambda i, j: (i, 0)),
            pl.BlockSpec((bn, k), lambda i, j: (j, 0)),
            pl.BlockSpec((1, bn), lambda i, j: (0, j)),
        ],
        out_specs=pl.BlockSpec((bm, bn), lambda i, j: (i, j)),
        out_shape=jax.ShapeDtypeStruct((m, n), jnp.float32),
        compiler_params=pltpu.CompilerParams(
            dimension_semantics=("parallel", "parallel"),
        ),
    )(a, w, b.reshape(1, n))


def _epi_kernel(v_ref, out_ref, idx_ref):
    v = v_ref[...]
    lanes = jax.lax.broadcasted_iota(jnp.int32, v.shape, 1)
    work = v
    vmax = None
    acc = jnp.zeros_like(v)
    ssum = jnp.zeros((v.shape[0], 1), jnp.float32)
    idx_cols = []
    for k in range(TOPK):
        m = jnp.max(work, axis=1, keepdims=True)
        is_m = work == m
        idx = jnp.min(jnp.where(is_m, lanes, v.shape[1]), axis=1, keepdims=True)
        if k == 0:
            vmax = m
        e = jnp.exp(m - vmax)
        onehot = lanes == idx
        acc = acc + jnp.where(onehot, e, 0.0)
        ssum = ssum + e
        idx_cols.append(idx)
        work = jnp.where(onehot, NEG_INF, work)
    out_ref[...] = acc / ssum
    idx_ref[...] = jnp.concatenate(idx_cols, axis=1)


@jax.jit
def _epilogue(v):
    m, e = v.shape
    br = 1024
    grid = (m // br,)
    return pl.pallas_call(
        _epi_kernel,
        grid=grid,
        in_specs=[
            pl.BlockSpec((br, e), lambda i: (i, 0)),
        ],
        out_specs=[
            pl.BlockSpec((br, e), lambda i: (i, 0)),
            pl.BlockSpec((br, TOPK), lambda i: (i, 0)),
        ],
        out_shape=[
            jax.ShapeDtypeStruct((m, e), jnp.float32),
            jax.ShapeDtypeStruct((m, TOPK), jnp.int32),
        ],
        compiler_params=pltpu.CompilerParams(
            dimension_semantics=("parallel",),
        ),
    )(v)


def kernel(x, W1, b1, W2, b2, Wn, bn, W3, b3, W4, b4, Wnl, bnl):
    h = _mm(x, W1, b1, True)
    h = _mm(h, W2, b2, True)
    h = _mm(h, Wn, bn, True)
    h = _mm(h, W3, b3, True)
    logits = _mm(h, W4, b4, False)
    noise_logits = _mm(x, Wnl, bnl, False)
    eps = jax.random.normal(jax.random.key(42), logits.shape, dtype=jnp.float32)
    noisy = logits + eps * jax.nn.softplus(noise_logits)
    return _epilogue(noisy)
